# baseline (speedup 1.0000x reference)
import functools

import jax
import jax.numpy as jnp
from jax import lax
from jax.experimental import pallas as pl
from jax.experimental.pallas import tpu as pltpu
from jax.experimental.pallas import tpu_sc as plsc

_S = 64.0
_M = 0.4

_CLS_BLK = 3072
_LANES = 16


def _scale_tile(x_ref, o_ref):
    o_ref[...] = x_ref[...] * _S


def _margin_body(b, rows_per_w, y_ref, labels_hbm, lab_v, idx_v, vals_v, sem):
    wid = lax.axis_index("s") * 2 + lax.axis_index("c")
    base = wid * rows_per_w
    pltpu.sync_copy(labels_hbm.at[pl.ds(base, rows_per_w)], lab_v)
    for k in range(rows_per_w // _LANES):
        l16 = lab_v[pl.ds(k * _LANES, _LANES)]
        row = lax.iota(jnp.int32, _LANES) + (base + k * _LANES)
        safe_l = jnp.where(l16 >= 0, l16, 0)
        idx16 = (
            (safe_l >> 3) * (8 * b)
            + (row >> 7) * 1024
            + (safe_l & 7) * 128
            + (row & 127)
        )
        idx_v[pl.ds(k * _LANES, _LANES)] = idx16
    pltpu.async_copy(y_ref.at[idx_v], vals_v, sem).wait()
    for k in range(rows_per_w // _LANES):
        v16 = vals_v[pl.ds(k * _LANES, _LANES)]
        l16 = lab_v[pl.ds(k * _LANES, _LANES)]
        vals_v[pl.ds(k * _LANES, _LANES)] = jnp.where(
            l16 >= 0, v16 - (_M * _S), v16)
    pltpu.async_copy(vals_v, y_ref.at[idx_v], sem).wait()


def kernel(logits, labels):
    b, c = logits.shape
    x_t = logits.T
    grid = (pl.cdiv(c, _CLS_BLK),)
    y_t = pl.pallas_call(
        _scale_tile,
        grid=grid,
        in_specs=[pl.BlockSpec((_CLS_BLK, b), lambda i: (i, 0))],
        out_specs=pl.BlockSpec((_CLS_BLK, b), lambda i: (i, 0)),
        out_shape=jax.ShapeDtypeStruct((c, b), logits.dtype),
    )(x_t)

    info = plsc.get_sparse_core_info()
    n_workers = info.num_cores * info.num_subcores
    rows_per_w = b // n_workers
    mesh = plsc.VectorSubcoreMesh(core_axis_name="c", subcore_axis_name="s")
    margin = pl.kernel(
        functools.partial(_margin_body, b, rows_per_w),
        out_type=(),
        mesh=mesh,
        scratch_types=[
            pltpu.VMEM((rows_per_w,), jnp.int32),
            pltpu.VMEM((rows_per_w,), jnp.int32),
            pltpu.VMEM((rows_per_w,), jnp.float32),
            pltpu.SemaphoreType.DMA,
        ],
    )
    z = (
        y_t.reshape(c // 8, 8, b // 128, 128)
        .transpose(0, 2, 1, 3)
        .reshape(c * b)
    )
    z_ref = jax.new_ref(z)
    margin(z_ref, labels.astype(jnp.int32))
    out_t = (
        z_ref[...]
        .reshape(c // 8, b // 128, 8, 128)
        .transpose(0, 2, 1, 3)
        .reshape(c, b)
    )
    return out_t.T
